# baseline (speedup 1.0000x reference)
import functools

import jax
import jax.numpy as jnp
from jax import lax
from jax.experimental import pallas as pl
from jax.experimental.pallas import tpu as pltpu
from jax.experimental.pallas import tpu_sc as plsc

_NBLK = 10000
_EBLK = 6400
_HPAD = 16
_SW = 8

_NC, _NS = 2, 16
_NW = _NC * _NS
_CH = 80
_NBUF = 5


def _sc_mesh():
    return plsc.VectorSubcoreMesh(core_axis_name="c", subcore_axis_name="s")


_SC_PARAMS = pltpu.CompilerParams(use_tc_tiling_on_sc=False)


def _sc_gather_add(streams):
    def build(i_src, i_dst, *tabs):
        e = i_src.shape[0]
        epw = e // _NW
        nch = epw // _CH
        nout = nch // _NBUF
        widths = [s[2] for s in streams]
        ns = len(widths)

        scr = [pltpu.VMEM((epw,), jnp.int32), pltpu.VMEM((epw,), jnp.int32)]
        for w in widths:
            scr.append(pltpu.VMEM((_NBUF, _CH, w), jnp.float32))
        for _ in range(3 * ns):
            scr.append(pltpu.SemaphoreType.DMA((_NBUF,)))

        @functools.partial(
            pl.kernel,
            out_type=tuple(
                jax.ShapeDtypeStruct((e, w), jnp.float32)
                for w in widths),
            mesh=_sc_mesh(),
            compiler_params=_SC_PARAMS,
            scratch_types=scr,
        )
        def k(is_hbm, id_hbm, *refs):
            tab = refs[:2 * ns]
            outs = refs[2 * ns:2 * ns + ns]
            isv = refs[2 * ns + ns]
            idv = refs[2 * ns + ns + 1]
            bufs = refs[2 * ns + ns + 2:2 * ns + ns + 2 + ns]
            sems = refs[2 * ns + ns + 2 + ns:]
            csem = sems[:ns]
            asem = sems[ns:2 * ns]
            ssem = sems[2 * ns:3 * ns]

            wid = lax.axis_index("s") * _NC + lax.axis_index("c")
            base = wid * epw
            pltpu.sync_copy(is_hbm.at[pl.ds(base, epw)], isv)
            pltpu.sync_copy(id_hbm.at[pl.ds(base, epw)], idv)

            def cop(s, b, ci):
                return (tab[2 * s].at[isv.at[pl.ds(ci * _CH, _CH)]],
                        bufs[s].at[b], csem[s].at[b])

            def acc(s, b, ci):
                return (tab[2 * s + 1].at[idv.at[pl.ds(ci * _CH, _CH)]],
                        bufs[s].at[b], asem[s].at[b])

            def sto(s, b, ci):
                return (bufs[s].at[b],
                        outs[s].at[pl.ds(base + ci * _CH, _CH)],
                        ssem[s].at[b])

            for b in range(_NBUF):
                for s in range(ns):
                    pltpu.async_copy(*cop(s, b, b))

            def outer(j, carry):
                for b in range(_NBUF):
                    ci = j * _NBUF + b
                    for s in range(ns):
                        pltpu.make_async_copy(*cop(s, b, ci)).wait()
                        pltpu.async_copy(*acc(s, b, ci), add=True)
                for b in range(_NBUF):
                    ci = j * _NBUF + b
                    for s in range(ns):
                        pltpu.make_async_copy(*acc(s, b, ci)).wait()
                        pltpu.async_copy(*sto(s, b, ci))
                for b in range(_NBUF):
                    ci = j * _NBUF + b

                    @pl.when(ci + _NBUF < nch)
                    def _():
                        for s in range(ns):
                            pltpu.make_async_copy(*sto(s, b, ci)).wait()
                            pltpu.async_copy(*cop(s, b, ci + _NBUF))
                return carry

            lax.fori_loop(0, nout, outer, 0)
            for b in range(_NBUF):
                ci = nch - _NBUF + b
                for s in range(ns):
                    pltpu.make_async_copy(*sto(s, b, ci)).wait()

        return k

    def run(i_src, i_dst):
        tabs = []
        for ts, td, _w in streams:
            tabs.append(ts)
            tabs.append(td)
        out = build(i_src, i_dst, *tabs)(i_src, i_dst, *tabs)
        return out if isinstance(out, (tuple, list)) else (out,)

    return run


def _sc_segsum(vals, dst, zeros):
    e = dst.shape[0]
    n = zeros.shape[0]
    epw = e // _NW
    nch = epw // _CH
    nout = nch // _NBUF

    @functools.partial(
        pl.kernel,
        out_type=jax.ShapeDtypeStruct((_NC, n, _SW), jnp.float32),
        mesh=_sc_mesh(),
        compiler_params=_SC_PARAMS,
        scratch_types=[
            pltpu.VMEM((epw,), jnp.int32),
            pltpu.VMEM((_NBUF, _CH, _SW), jnp.float32),
            pltpu.VMEM_SHARED((n, _SW), jnp.float32),
            pltpu.SemaphoreType.DMA((_NBUF,)),
            pltpu.SemaphoreType.DMA((_NBUF,)),
        ],
    )
    def k(vr, dst_hbm, zr, outr, idx_v, vb, shared, vsem, asem):
        cid = lax.axis_index("c")
        sid = lax.axis_index("s")
        wid = sid * _NC + cid
        base = wid * epw
        pltpu.sync_copy(dst_hbm.at[pl.ds(base, epw)], idx_v)

        @pl.when(sid == 0)
        def _():
            pltpu.sync_copy(zr, shared)

        plsc.subcore_barrier()

        def v(b, ci):
            return (vr.at[pl.ds(base + ci * _CH, _CH)], vb.at[b],
                    vsem.at[b])

        def a(b, ci):
            return (vb.at[b], shared.at[idx_v.at[pl.ds(ci * _CH, _CH)]],
                    asem.at[b])

        for b in range(_NBUF):
            pltpu.async_copy(*v(b, b))

        def outer(j, carry):
            for b in range(_NBUF):
                ci = j * _NBUF + b
                pltpu.make_async_copy(*v(b, ci)).wait()
                pltpu.async_copy(*a(b, ci), add=True)
            for b in range(_NBUF):
                ci = j * _NBUF + b

                @pl.when(ci + _NBUF < nch)
                def _():
                    pltpu.make_async_copy(*a(b, ci)).wait()
                    pltpu.async_copy(*v(b, ci + _NBUF))
            return carry

        lax.fori_loop(0, nout, outer, 0)
        for b in range(_NBUF):
            pltpu.make_async_copy(*a(b, nch - _NBUF + b)).wait()

        plsc.subcore_barrier()

        @pl.when(sid == 0)
        def _():
            pltpu.sync_copy(shared, outr.at[cid])

    return k(vals, dst, zeros)


def _call(body, grid, in_arrays, in_specs, out_shape, out_specs):
    return pl.pallas_call(
        body, grid=(grid,), in_specs=in_specs, out_specs=out_specs,
        out_shape=out_shape)(*in_arrays)


def _full_spec(a):
    return pl.BlockSpec(a.shape, lambda i, _nd=a.ndim: (0,) * _nd)


def _row_spec(rows, cols):
    return pl.BlockSpec((rows, cols), lambda i: (i, 0))


def _dot(a, b):
    return jnp.dot(a, b, preferred_element_type=jnp.float32)


def _padcols(v, w):
    return jnp.concatenate(
        [v, jnp.zeros((v.shape[0], w - v.shape[1]), jnp.float32)], axis=1)


def _b2d(b):
    return b.reshape(1, -1)


def _pk(v):
    return v.reshape(v.shape[0] * v.shape[1] // 128, 128)


def _upk(ref, w):
    v = ref[...]
    return v.reshape(v.shape[0] * 128 // w, w)


def _node_pre_body(x_ref, w1s_ref, w1d_ref, wn1_ref, wn2_ref,
                   v1s_ref, v1d_ref,
                   t1_ref, t2_ref, hp_ref, q1s_ref, q1d_ref):
    x = x_ref[...]
    t1_ref[...] = _dot(x, w1s_ref[...])
    t2_ref[...] = _dot(x, w1d_ref[...])
    t = jax.nn.relu(_dot(x, wn1_ref[...]))
    h = jax.nn.relu(_dot(t, wn2_ref[...]))
    hp_ref[...] = _padcols(h, _HPAD)
    q1s_ref[...] = _dot(h, v1s_ref[...])
    q1d_ref[...] = _dot(h, v1d_ref[...])


def _ec_body(g64_ref, g40_ref, ea_ref,
             w1e_ref, b1_ref, w2_ref, b2_ref, w3_ref, b3_ref,
             we1_ref, we2_ref, v1e_ref, c1_ref, v2_ref, c2_ref,
             w_ref, mf_ref, e1_ref, es1_ref):
    g64 = g64_ref[...]
    g40 = g40_ref[...]
    ea = ea_ref[...]
    h1 = jax.nn.relu(g64 + _dot(ea, w1e_ref[...]) + b1_ref[...])
    h2 = jax.nn.relu(_dot(h1, w2_ref[...]) + b2_ref[...])
    logit = _dot(h2, w3_ref[...]) + b3_ref[...]
    wv = jax.nn.sigmoid(logit)
    mf = (wv > 0.5).astype(jnp.float32)
    w_ref[...] = _padcols(wv, _SW)
    mf_ref[...] = _padcols(mf, _SW)
    t = jax.nn.relu(_dot(ea, we1_ref[...]))
    e0 = jax.nn.relu(_dot(t, we2_ref[...]))
    m = jax.nn.relu(g40 + _dot(e0, v1e_ref[...]) + c1_ref[...])
    e1 = jax.nn.relu(_dot(m, v2_ref[...]) + c2_ref[...])
    e1_ref[...] = e1
    es1_ref[...] = _padcols(e1 * mf, _SW)


def _resin_edge_body(g40_ref, e_ref, mf_ref, v1e_ref, c1_ref,
                     v2_ref, c2_ref, enew_ref, escat_ref):
    g40 = g40_ref[...]
    ev = e_ref[...]
    mf = mf_ref[:, :1]
    m = jax.nn.relu(g40 + _dot(ev, v1e_ref[...]) + c1_ref[...])
    e_new = jax.nn.relu(_dot(m, v2_ref[...]) + c2_ref[...])
    enew_ref[...] = e_new
    escat_ref[...] = _padcols(e_new * mf, _SW)


def _resin_node_body(hp_ref, a0_ref, a1_ref, p1_ref, d1_ref, p2_ref,
                     d2_ref, vs_ref, vd_ref, hpn_ref, qs_ref, qd_ref):
    h = hp_ref[:, :5]
    agg = a0_ref[:, :4] + a1_ref[:, :4]
    cat = jnp.concatenate([h, agg], axis=1)
    m = jax.nn.relu(_dot(cat, p1_ref[...]) + d1_ref[...])
    h_new = _dot(m, p2_ref[...]) + d2_ref[...]
    hn = jax.nn.relu(h + h_new)
    hpn_ref[...] = _padcols(hn, _HPAD)
    qs_ref[...] = _dot(hn, vs_ref[...])
    qd_ref[...] = _dot(hn, vd_ref[...])


def _tp_edge_body(g40_ref, e1_ref, e2_ref, mf_ref,
                  t1e_ref, tb1_ref, t2_ref, tb2_ref, tpscat_ref):
    g40 = g40_ref[...]
    e1 = e1_ref[...]
    e2 = e2_ref[...]
    mf = mf_ref[:, :1]
    ecat = jnp.concatenate([e1, e2], axis=1)
    m = jax.nn.relu(g40 + _dot(ecat, t1e_ref[...]) + tb1_ref[...])
    tp = _dot(m, t2_ref[...]) + tb2_ref[...]
    tpscat_ref[...] = _padcols(tp * mf, _SW)


def _heads_body(hp_ref, a0_ref, a1_ref,
                pb1_ref, bb1_ref, pb2_ref, bb2_ref, pb3_ref, bb3_ref,
                pc1_ref, cb1_ref, pc2_ref, cb2_ref, pc3_ref, cb3_ref,
                tn1_ref, nb1_ref, tn2_ref, nb2_ref,
                beta_ref, hh_ref, pp_ref):
    h = hp_ref[:, :5]
    agg2 = a0_ref[:, :1] + a1_ref[:, :1]
    tb = jax.nn.relu(_dot(h, pb1_ref[...]) + bb1_ref[...])
    tb = jax.nn.relu(_dot(tb, pb2_ref[...]) + bb2_ref[...])
    beta_ref[...] = jax.nn.sigmoid(_dot(tb, pb3_ref[...]) + bb3_ref[...]) + 1e-8
    tc = jax.nn.relu(_dot(h, pc1_ref[...]) + cb1_ref[...])
    tc = jax.nn.relu(_dot(tc, pc2_ref[...]) + cb2_ref[...])
    hh_ref[...] = _dot(tc, pc3_ref[...]) + cb3_ref[...]
    cat = jnp.concatenate([h, agg2], axis=1)
    tn = jax.nn.relu(_dot(cat, tn1_ref[...]) + nb1_ref[...])
    pp_ref[...] = _dot(tn, tn2_ref[...]) + nb2_ref[...]


def kernel(x, edge_index, edge_attr, params):
    n = x.shape[0]
    e = edge_index.shape[1]
    src = edge_index[0]
    dst = edge_index[1]
    zer = jnp.zeros((n, _SW), jnp.float32)
    eg = e // _EBLK
    ng = n // _NBLK

    ec = params["ec"]
    w1, b1 = ec[0]
    w1s, w1d, w1e = w1[:128], w1[128:256], w1[256:260]
    w2, b2 = ec[1]
    w3, b3 = ec[2]
    wn1 = params["node_enc"][0][0]
    wn2 = params["node_enc"][1][0]
    we1 = params["edge_enc"][0][0]
    we2 = params["edge_enc"][1][0]
    rl = params["resin"]
    rv1 = [rl[i]["phi_e"][0][0] for i in range(2)]
    rc1 = [rl[i]["phi_e"][0][1] for i in range(2)]
    rv2 = [rl[i]["phi_e"][1][0] for i in range(2)]
    rc2 = [rl[i]["phi_e"][1][1] for i in range(2)]
    rp1 = [rl[i]["phi_n"][0][0] for i in range(2)]
    rd1 = [rl[i]["phi_n"][0][1] for i in range(2)]
    rp2 = [rl[i]["phi_n"][1][0] for i in range(2)]
    rd2 = [rl[i]["phi_n"][1][1] for i in range(2)]
    t1, tb1 = params["tp_phi_e"][0]
    t2, tb2 = params["tp_phi_e"][1]

    aw = [w1s, w1d, wn1, wn2, rv1[0][:5], rv1[0][5:10]]
    tab1, tab2, hp, q1s, q1d = _call(
        _node_pre_body, ng,
        [x] + aw,
        [_row_spec(_NBLK, 128)] + [_full_spec(a) for a in aw],
        [jax.ShapeDtypeStruct((n, 64), jnp.float32),
         jax.ShapeDtypeStruct((n, 64), jnp.float32),
         jax.ShapeDtypeStruct((n, _HPAD), jnp.float32),
         jax.ShapeDtypeStruct((n, 40), jnp.float32),
         jax.ShapeDtypeStruct((n, 40), jnp.float32)],
        [_row_spec(_NBLK, 64), _row_spec(_NBLK, 64),
         _row_spec(_NBLK, _HPAD),
         _row_spec(_NBLK, 40), _row_spec(_NBLK, 40)])

    g128, g40 = _sc_gather_add([(tab1, tab2, 64), (q1s, q1d, 40)])(src, dst)

    ecw = [w1e, _b2d(b1), w2, _b2d(b2), w3, _b2d(b3), we1, we2,
           rv1[0][10:14], _b2d(rc1[0]), rv2[0], _b2d(rc2[0])]
    w_arr, mf_arr, e1_arr, es1 = _call(
        _ec_body, eg,
        [g128, g40, edge_attr] + ecw,
        [_row_spec(_EBLK, 64),
         _row_spec(_EBLK, 40),
         _row_spec(_EBLK, 4)]
        + [_full_spec(a) for a in ecw],
        [jax.ShapeDtypeStruct((e, _SW), jnp.float32),
         jax.ShapeDtypeStruct((e, _SW), jnp.float32),
         jax.ShapeDtypeStruct((e, 4), jnp.float32),
         jax.ShapeDtypeStruct((e, _SW), jnp.float32)],
        [_row_spec(_EBLK, _SW),
         _row_spec(_EBLK, _SW),
         _row_spec(_EBLK, 4),
         _row_spec(_EBLK, _SW)])

    agg = _sc_segsum(es1, dst, zer)
    nw1 = [rp1[0], _b2d(rd1[0]), rp2[0], _b2d(rd2[0]),
           rv1[1][:5], rv1[1][5:10]]
    hp, q2s, q2d = _call(
        _resin_node_body, ng,
        [hp, agg[0], agg[1]] + nw1,
        [_row_spec(_NBLK, _HPAD), _row_spec(_NBLK, _SW),
         _row_spec(_NBLK, _SW)]
        + [_full_spec(a) for a in nw1],
        [jax.ShapeDtypeStruct((n, _HPAD), jnp.float32),
         jax.ShapeDtypeStruct((n, 40), jnp.float32),
         jax.ShapeDtypeStruct((n, 40), jnp.float32)],
        [_row_spec(_NBLK, _HPAD),
         _row_spec(_NBLK, 40), _row_spec(_NBLK, 40)])

    (g40b,) = _sc_gather_add([(q2s, q2d, 40)])(src, dst)
    ew2 = [rv1[1][10:14], _b2d(rc1[1]), rv2[1], _b2d(rc2[1])]
    e2_arr, es2 = _call(
        _resin_edge_body, eg,
        [g40b, e1_arr, mf_arr] + ew2,
        [_row_spec(_EBLK, 40),
         _row_spec(_EBLK, 4),
         _row_spec(_EBLK, _SW)]
        + [_full_spec(a) for a in ew2],
        [jax.ShapeDtypeStruct((e, 4), jnp.float32),
         jax.ShapeDtypeStruct((e, _SW), jnp.float32)],
        [_row_spec(_EBLK, 4),
         _row_spec(_EBLK, _SW)])
    agg = _sc_segsum(es2, dst, zer)
    nw2 = [rp1[1], _b2d(rd1[1]), rp2[1], _b2d(rd2[1]),
           t1[:5], t1[5:10]]
    hp, qts, qtd = _call(
        _resin_node_body, ng,
        [hp, agg[0], agg[1]] + nw2,
        [_row_spec(_NBLK, _HPAD), _row_spec(_NBLK, _SW),
         _row_spec(_NBLK, _SW)]
        + [_full_spec(a) for a in nw2],
        [jax.ShapeDtypeStruct((n, _HPAD), jnp.float32),
         jax.ShapeDtypeStruct((n, 40), jnp.float32),
         jax.ShapeDtypeStruct((n, 40), jnp.float32)],
        [_row_spec(_NBLK, _HPAD),
         _row_spec(_NBLK, 40), _row_spec(_NBLK, 40)])

    (gt,) = _sc_gather_add([(qts, qtd, 40)])(src, dst)
    tpw = [t1[10:18], _b2d(tb1), t2, _b2d(tb2)]
    tp_arr = _call(
        _tp_edge_body, eg,
        [gt, e1_arr, e2_arr, mf_arr] + tpw,
        [_row_spec(_EBLK, 40),
         _row_spec(_EBLK, 4),
         _row_spec(_EBLK, 4),
         _row_spec(_EBLK, _SW)]
        + [_full_spec(a) for a in tpw],
        [jax.ShapeDtypeStruct((e, _SW), jnp.float32)],
        [_row_spec(_EBLK, _SW)])[0]
    agg2 = _sc_segsum(tp_arr, dst, zer)

    pb = params["p_beta"]
    pc = params["p_cluster"]
    tn = params["tp_phi_n"]
    wts = [pb[0][0], _b2d(pb[0][1]), pb[1][0], _b2d(pb[1][1]),
           pb[2][0], _b2d(pb[2][1]),
           pc[0][0], _b2d(pc[0][1]), pc[1][0], _b2d(pc[1][1]),
           pc[2][0], _b2d(pc[2][1]),
           tn[0][0], _b2d(tn[0][1]), tn[1][0], _b2d(tn[1][1])]
    beta, hh, pp = _call(
        _heads_body, ng,
        [hp, agg2[0], agg2[1]] + wts,
        [_row_spec(_NBLK, _HPAD), _row_spec(_NBLK, _SW),
         _row_spec(_NBLK, _SW)]
        + [_full_spec(a) for a in wts],
        [jax.ShapeDtypeStruct((n, 1), jnp.float32),
         jax.ShapeDtypeStruct((n, 2), jnp.float32),
         jax.ShapeDtypeStruct((n, 1), jnp.float32)],
        [_row_spec(_NBLK, 1), _row_spec(_NBLK, 2), _row_spec(_NBLK, 1)])

    w_flat = w_arr[:, 0]
    hit_mask = jnp.ones((n,), dtype=bool)
    edge_mask = w_flat > 0.5
    return (w_flat.reshape(e, 1), hh, beta, pp, hit_mask, edge_mask)


# speedup vs baseline: 1.0210x; 1.0210x over previous
import functools

import jax
import jax.numpy as jnp
from jax import lax
from jax.experimental import pallas as pl
from jax.experimental.pallas import tpu as pltpu
from jax.experimental.pallas import tpu_sc as plsc

_NBLK = 10000
_EBLK = 6400
_HPAD = 16
_SW = 8

_NC, _NS = 2, 16
_NW = _NC * _NS
_CH = 80
_NBUF = 5


def _sc_mesh():
    return plsc.VectorSubcoreMesh(core_axis_name="c", subcore_axis_name="s")


_SC_PARAMS = pltpu.CompilerParams(use_tc_tiling_on_sc=False)


def _sc_gather_add(streams):
    def build(i_src, i_dst, *tabs):
        e = i_src.shape[0]
        epw = e // _NW
        nch = epw // _CH
        nout = nch // _NBUF
        widths = [s[2] for s in streams]
        ns = len(widths)

        scr = [pltpu.VMEM((epw,), jnp.int32), pltpu.VMEM((epw,), jnp.int32)]
        for w in widths:
            scr.append(pltpu.VMEM((_NBUF, _CH, w), jnp.float32))
        for _ in range(3 * ns):
            scr.append(pltpu.SemaphoreType.DMA((_NBUF,)))

        @functools.partial(
            pl.kernel,
            out_type=tuple(
                jax.ShapeDtypeStruct((e, w), jnp.float32)
                for w in widths),
            mesh=_sc_mesh(),
            compiler_params=_SC_PARAMS,
            scratch_types=scr,
        )
        def k(is_hbm, id_hbm, *refs):
            tab = refs[:2 * ns]
            outs = refs[2 * ns:2 * ns + ns]
            isv = refs[2 * ns + ns]
            idv = refs[2 * ns + ns + 1]
            bufs = refs[2 * ns + ns + 2:2 * ns + ns + 2 + ns]
            sems = refs[2 * ns + ns + 2 + ns:]
            csem = sems[:ns]
            asem = sems[ns:2 * ns]
            ssem = sems[2 * ns:3 * ns]

            wid = lax.axis_index("s") * _NC + lax.axis_index("c")
            base = wid * epw
            pltpu.sync_copy(is_hbm.at[pl.ds(base, epw)], isv)
            pltpu.sync_copy(id_hbm.at[pl.ds(base, epw)], idv)

            def cop(s, b, ci):
                return (tab[2 * s].at[isv.at[pl.ds(ci * _CH, _CH)]],
                        bufs[s].at[b], csem[s].at[b])

            def acc(s, b, ci):
                return (tab[2 * s + 1].at[idv.at[pl.ds(ci * _CH, _CH)]],
                        bufs[s].at[b], asem[s].at[b])

            def sto(s, b, ci):
                return (bufs[s].at[b],
                        outs[s].at[pl.ds(base + ci * _CH, _CH)],
                        ssem[s].at[b])

            for b in range(_NBUF):
                for s in range(ns):
                    pltpu.async_copy(*cop(s, b, b))

            def outer(j, carry):
                for b in range(_NBUF):
                    ci = j * _NBUF + b
                    for s in range(ns):
                        pltpu.make_async_copy(*cop(s, b, ci)).wait()
                        pltpu.async_copy(*acc(s, b, ci), add=True)
                for b in range(_NBUF):
                    ci = j * _NBUF + b
                    for s in range(ns):
                        pltpu.make_async_copy(*acc(s, b, ci)).wait()
                        pltpu.async_copy(*sto(s, b, ci))
                for b in range(_NBUF):
                    ci = j * _NBUF + b

                    @pl.when(ci + _NBUF < nch)
                    def _():
                        for s in range(ns):
                            pltpu.make_async_copy(*sto(s, b, ci)).wait()
                            pltpu.async_copy(*cop(s, b, ci + _NBUF))
                return carry

            lax.fori_loop(0, nout, outer, 0)
            for b in range(_NBUF):
                ci = nch - _NBUF + b
                for s in range(ns):
                    pltpu.make_async_copy(*sto(s, b, ci)).wait()

        return k

    def run(i_src, i_dst):
        tabs = []
        for ts, td, _w in streams:
            tabs.append(ts)
            tabs.append(td)
        out = build(i_src, i_dst, *tabs)(i_src, i_dst, *tabs)
        return out if isinstance(out, (tuple, list)) else (out,)

    return run


def _sc_segsum(vals, dst, zeros):
    e = dst.shape[0]
    n = zeros.shape[0]
    epw = e // _NW
    nch = epw // _CH
    nout = nch // _NBUF

    @functools.partial(
        pl.kernel,
        out_type=jax.ShapeDtypeStruct((_NC, n, _SW), jnp.float32),
        mesh=_sc_mesh(),
        compiler_params=_SC_PARAMS,
        scratch_types=[
            pltpu.VMEM((epw,), jnp.int32),
            pltpu.VMEM((_NBUF, _CH, _SW), jnp.float32),
            pltpu.VMEM_SHARED((n, _SW), jnp.float32),
            pltpu.SemaphoreType.DMA((_NBUF,)),
            pltpu.SemaphoreType.DMA((_NBUF,)),
        ],
    )
    def k(vr, dst_hbm, zr, outr, idx_v, vb, shared, vsem, asem):
        cid = lax.axis_index("c")
        sid = lax.axis_index("s")
        wid = sid * _NC + cid
        base = wid * epw
        pltpu.sync_copy(dst_hbm.at[pl.ds(base, epw)], idx_v)

        @pl.when(sid == 0)
        def _():
            pltpu.sync_copy(zr, shared)

        plsc.subcore_barrier()

        def v(b, ci):
            return (vr.at[pl.ds(base + ci * _CH, _CH)], vb.at[b],
                    vsem.at[b])

        def a(b, ci):
            return (vb.at[b], shared.at[idx_v.at[pl.ds(ci * _CH, _CH)]],
                    asem.at[b])

        for b in range(_NBUF):
            pltpu.async_copy(*v(b, b))

        def outer(j, carry):
            for b in range(_NBUF):
                ci = j * _NBUF + b
                pltpu.make_async_copy(*v(b, ci)).wait()
                pltpu.async_copy(*a(b, ci), add=True)
            for b in range(_NBUF):
                ci = j * _NBUF + b

                @pl.when(ci + _NBUF < nch)
                def _():
                    pltpu.make_async_copy(*a(b, ci)).wait()
                    pltpu.async_copy(*v(b, ci + _NBUF))
            return carry

        lax.fori_loop(0, nout, outer, 0)
        for b in range(_NBUF):
            pltpu.make_async_copy(*a(b, nch - _NBUF + b)).wait()

        plsc.subcore_barrier()

        @pl.when(sid == 0)
        def _():
            pltpu.sync_copy(shared, outr.at[cid])

    return k(vals, dst, zeros)


def _call(body, grid, in_arrays, in_specs, out_shape, out_specs):
    return pl.pallas_call(
        body, grid=(grid,), in_specs=in_specs, out_specs=out_specs,
        out_shape=out_shape)(*in_arrays)


def _full_spec(a):
    return pl.BlockSpec(a.shape, lambda i, _nd=a.ndim: (0,) * _nd)


def _row_spec(rows, cols):
    return pl.BlockSpec((rows, cols), lambda i: (i, 0))


def _dot(a, b):
    return jnp.dot(a, b, preferred_element_type=jnp.float32)


def _padcols(v, w):
    return jnp.concatenate(
        [v, jnp.zeros((v.shape[0], w - v.shape[1]), jnp.float32)], axis=1)


def _b2d(b):
    return b.reshape(1, -1)


def _pk(v):
    return v.reshape(v.shape[0] * v.shape[1] // 128, 128)


def _upk(ref, w):
    v = ref[...]
    return v.reshape(v.shape[0] * 128 // w, w)


def _node_pre_body(x_ref, w1s_ref, w1d_ref, wn1_ref, wn2_ref,
                   v1s_ref, v1d_ref,
                   t1_ref, t2_ref, hp_ref, q1s_ref, q1d_ref):
    x = x_ref[...]
    t1_ref[...] = _dot(x, w1s_ref[...])
    t2_ref[...] = _dot(x, w1d_ref[...])
    t = jax.nn.relu(_dot(x, wn1_ref[...]))
    h = jax.nn.relu(_dot(t, wn2_ref[...]))
    hp_ref[...] = _padcols(h, _HPAD)
    q1s_ref[...] = _dot(h, v1s_ref[...])
    q1d_ref[...] = _dot(h, v1d_ref[...])


def _ec_body(g64_ref, g40_ref, ea_ref,
             w1e_ref, b1_ref, w2_ref, b2_ref, w3_ref, b3_ref,
             we1_ref, we2_ref, v1e_ref, c1_ref, v2_ref, c2_ref,
             w_ref, e1_ref, es1_ref):
    g64 = g64_ref[...]
    g40 = g40_ref[...]
    ea = ea_ref[...]
    h1 = jax.nn.relu(g64 + _dot(ea, w1e_ref[...]) + b1_ref[...])
    h2 = jax.nn.relu(_dot(h1, w2_ref[...]) + b2_ref[...])
    logit = _dot(h2, w3_ref[...]) + b3_ref[...]
    wv = jax.nn.sigmoid(logit)
    mf = (wv > 0.5).astype(jnp.float32)
    w_ref[...] = _padcols(wv, _SW)
    t = jax.nn.relu(_dot(ea, we1_ref[...]))
    e0 = jax.nn.relu(_dot(t, we2_ref[...]))
    m = jax.nn.relu(g40 + _dot(e0, v1e_ref[...]) + c1_ref[...])
    e1 = jax.nn.relu(_dot(m, v2_ref[...]) + c2_ref[...])
    e1_ref[...] = e1
    es1_ref[...] = _padcols(e1 * mf, _SW)


def _resin_edge_body(g40_ref, e_ref, w_ref, v1e_ref, c1_ref,
                     v2_ref, c2_ref, enew_ref, escat_ref):
    g40 = g40_ref[...]
    ev = e_ref[...]
    mf = (w_ref[:, :1] > 0.5).astype(jnp.float32)
    m = jax.nn.relu(g40 + _dot(ev, v1e_ref[...]) + c1_ref[...])
    e_new = jax.nn.relu(_dot(m, v2_ref[...]) + c2_ref[...])
    enew_ref[...] = e_new
    escat_ref[...] = _padcols(e_new * mf, _SW)


def _resin_node_body(hp_ref, a0_ref, a1_ref, p1_ref, d1_ref, p2_ref,
                     d2_ref, vs_ref, vd_ref, hpn_ref, qs_ref, qd_ref):
    h = hp_ref[:, :5]
    agg = a0_ref[:, :4] + a1_ref[:, :4]
    cat = jnp.concatenate([h, agg], axis=1)
    m = jax.nn.relu(_dot(cat, p1_ref[...]) + d1_ref[...])
    h_new = _dot(m, p2_ref[...]) + d2_ref[...]
    hn = jax.nn.relu(h + h_new)
    hpn_ref[...] = _padcols(hn, _HPAD)
    qs_ref[...] = _dot(hn, vs_ref[...])
    qd_ref[...] = _dot(hn, vd_ref[...])


def _tp_edge_body(g40_ref, e1_ref, e2_ref, w_ref,
                  t1e_ref, tb1_ref, t2_ref, tb2_ref, tpscat_ref):
    g40 = g40_ref[...]
    e1 = e1_ref[...]
    e2 = e2_ref[...]
    mf = (w_ref[:, :1] > 0.5).astype(jnp.float32)
    ecat = jnp.concatenate([e1, e2], axis=1)
    m = jax.nn.relu(g40 + _dot(ecat, t1e_ref[...]) + tb1_ref[...])
    tp = _dot(m, t2_ref[...]) + tb2_ref[...]
    tpscat_ref[...] = _padcols(tp * mf, _SW)


def _heads_body(hp_ref, a0_ref, a1_ref,
                pb1_ref, bb1_ref, pb2_ref, bb2_ref, pb3_ref, bb3_ref,
                pc1_ref, cb1_ref, pc2_ref, cb2_ref, pc3_ref, cb3_ref,
                tn1_ref, nb1_ref, tn2_ref, nb2_ref,
                beta_ref, hh_ref, pp_ref):
    h = hp_ref[:, :5]
    agg2 = a0_ref[:, :1] + a1_ref[:, :1]
    tb = jax.nn.relu(_dot(h, pb1_ref[...]) + bb1_ref[...])
    tb = jax.nn.relu(_dot(tb, pb2_ref[...]) + bb2_ref[...])
    beta_ref[...] = jax.nn.sigmoid(_dot(tb, pb3_ref[...]) + bb3_ref[...]) + 1e-8
    tc = jax.nn.relu(_dot(h, pc1_ref[...]) + cb1_ref[...])
    tc = jax.nn.relu(_dot(tc, pc2_ref[...]) + cb2_ref[...])
    hh_ref[...] = _dot(tc, pc3_ref[...]) + cb3_ref[...]
    cat = jnp.concatenate([h, agg2], axis=1)
    tn = jax.nn.relu(_dot(cat, tn1_ref[...]) + nb1_ref[...])
    pp_ref[...] = _dot(tn, tn2_ref[...]) + nb2_ref[...]


def kernel(x, edge_index, edge_attr, params):
    n = x.shape[0]
    e = edge_index.shape[1]
    src = edge_index[0]
    dst = edge_index[1]
    zer = jnp.zeros((n, _SW), jnp.float32)
    eg = e // _EBLK
    ng = n // _NBLK

    ec = params["ec"]
    w1, b1 = ec[0]
    w1s, w1d, w1e = w1[:128], w1[128:256], w1[256:260]
    w2, b2 = ec[1]
    w3, b3 = ec[2]
    wn1 = params["node_enc"][0][0]
    wn2 = params["node_enc"][1][0]
    we1 = params["edge_enc"][0][0]
    we2 = params["edge_enc"][1][0]
    rl = params["resin"]
    rv1 = [rl[i]["phi_e"][0][0] for i in range(2)]
    rc1 = [rl[i]["phi_e"][0][1] for i in range(2)]
    rv2 = [rl[i]["phi_e"][1][0] for i in range(2)]
    rc2 = [rl[i]["phi_e"][1][1] for i in range(2)]
    rp1 = [rl[i]["phi_n"][0][0] for i in range(2)]
    rd1 = [rl[i]["phi_n"][0][1] for i in range(2)]
    rp2 = [rl[i]["phi_n"][1][0] for i in range(2)]
    rd2 = [rl[i]["phi_n"][1][1] for i in range(2)]
    t1, tb1 = params["tp_phi_e"][0]
    t2, tb2 = params["tp_phi_e"][1]

    aw = [w1s, w1d, wn1, wn2, rv1[0][:5], rv1[0][5:10]]
    tab1, tab2, hp, q1s, q1d = _call(
        _node_pre_body, ng,
        [x] + aw,
        [_row_spec(_NBLK, 128)] + [_full_spec(a) for a in aw],
        [jax.ShapeDtypeStruct((n, 64), jnp.float32),
         jax.ShapeDtypeStruct((n, 64), jnp.float32),
         jax.ShapeDtypeStruct((n, _HPAD), jnp.float32),
         jax.ShapeDtypeStruct((n, 40), jnp.float32),
         jax.ShapeDtypeStruct((n, 40), jnp.float32)],
        [_row_spec(_NBLK, 64), _row_spec(_NBLK, 64),
         _row_spec(_NBLK, _HPAD),
         _row_spec(_NBLK, 40), _row_spec(_NBLK, 40)])

    g128, g40 = _sc_gather_add([(tab1, tab2, 64), (q1s, q1d, 40)])(src, dst)

    ecw = [w1e, _b2d(b1), w2, _b2d(b2), w3, _b2d(b3), we1, we2,
           rv1[0][10:14], _b2d(rc1[0]), rv2[0], _b2d(rc2[0])]
    w_arr, e1_arr, es1 = _call(
        _ec_body, eg,
        [g128, g40, edge_attr] + ecw,
        [_row_spec(_EBLK, 64),
         _row_spec(_EBLK, 40),
         _row_spec(_EBLK, 4)]
        + [_full_spec(a) for a in ecw],
        [jax.ShapeDtypeStruct((e, _SW), jnp.float32),
         jax.ShapeDtypeStruct((e, 4), jnp.float32),
         jax.ShapeDtypeStruct((e, _SW), jnp.float32)],
        [_row_spec(_EBLK, _SW),
         _row_spec(_EBLK, 4),
         _row_spec(_EBLK, _SW)])

    agg = _sc_segsum(es1, dst, zer)
    nw1 = [rp1[0], _b2d(rd1[0]), rp2[0], _b2d(rd2[0]),
           rv1[1][:5], rv1[1][5:10]]
    hp, q2s, q2d = _call(
        _resin_node_body, ng,
        [hp, agg[0], agg[1]] + nw1,
        [_row_spec(_NBLK, _HPAD), _row_spec(_NBLK, _SW),
         _row_spec(_NBLK, _SW)]
        + [_full_spec(a) for a in nw1],
        [jax.ShapeDtypeStruct((n, _HPAD), jnp.float32),
         jax.ShapeDtypeStruct((n, 40), jnp.float32),
         jax.ShapeDtypeStruct((n, 40), jnp.float32)],
        [_row_spec(_NBLK, _HPAD),
         _row_spec(_NBLK, 40), _row_spec(_NBLK, 40)])

    (g40b,) = _sc_gather_add([(q2s, q2d, 40)])(src, dst)
    ew2 = [rv1[1][10:14], _b2d(rc1[1]), rv2[1], _b2d(rc2[1])]
    e2_arr, es2 = _call(
        _resin_edge_body, eg,
        [g40b, e1_arr, w_arr] + ew2,
        [_row_spec(_EBLK, 40),
         _row_spec(_EBLK, 4),
         _row_spec(_EBLK, _SW)]
        + [_full_spec(a) for a in ew2],
        [jax.ShapeDtypeStruct((e, 4), jnp.float32),
         jax.ShapeDtypeStruct((e, _SW), jnp.float32)],
        [_row_spec(_EBLK, 4),
         _row_spec(_EBLK, _SW)])
    agg = _sc_segsum(es2, dst, zer)
    nw2 = [rp1[1], _b2d(rd1[1]), rp2[1], _b2d(rd2[1]),
           t1[:5], t1[5:10]]
    hp, qts, qtd = _call(
        _resin_node_body, ng,
        [hp, agg[0], agg[1]] + nw2,
        [_row_spec(_NBLK, _HPAD), _row_spec(_NBLK, _SW),
         _row_spec(_NBLK, _SW)]
        + [_full_spec(a) for a in nw2],
        [jax.ShapeDtypeStruct((n, _HPAD), jnp.float32),
         jax.ShapeDtypeStruct((n, 40), jnp.float32),
         jax.ShapeDtypeStruct((n, 40), jnp.float32)],
        [_row_spec(_NBLK, _HPAD),
         _row_spec(_NBLK, 40), _row_spec(_NBLK, 40)])

    (gt,) = _sc_gather_add([(qts, qtd, 40)])(src, dst)
    tpw = [t1[10:18], _b2d(tb1), t2, _b2d(tb2)]
    tp_arr = _call(
        _tp_edge_body, eg,
        [gt, e1_arr, e2_arr, w_arr] + tpw,
        [_row_spec(_EBLK, 40),
         _row_spec(_EBLK, 4),
         _row_spec(_EBLK, 4),
         _row_spec(_EBLK, _SW)]
        + [_full_spec(a) for a in tpw],
        [jax.ShapeDtypeStruct((e, _SW), jnp.float32)],
        [_row_spec(_EBLK, _SW)])[0]
    agg2 = _sc_segsum(tp_arr, dst, zer)

    pb = params["p_beta"]
    pc = params["p_cluster"]
    tn = params["tp_phi_n"]
    wts = [pb[0][0], _b2d(pb[0][1]), pb[1][0], _b2d(pb[1][1]),
           pb[2][0], _b2d(pb[2][1]),
           pc[0][0], _b2d(pc[0][1]), pc[1][0], _b2d(pc[1][1]),
           pc[2][0], _b2d(pc[2][1]),
           tn[0][0], _b2d(tn[0][1]), tn[1][0], _b2d(tn[1][1])]
    beta, hh, pp = _call(
        _heads_body, ng,
        [hp, agg2[0], agg2[1]] + wts,
        [_row_spec(_NBLK, _HPAD), _row_spec(_NBLK, _SW),
         _row_spec(_NBLK, _SW)]
        + [_full_spec(a) for a in wts],
        [jax.ShapeDtypeStruct((n, 1), jnp.float32),
         jax.ShapeDtypeStruct((n, 2), jnp.float32),
         jax.ShapeDtypeStruct((n, 1), jnp.float32)],
        [_row_spec(_NBLK, 1), _row_spec(_NBLK, 2), _row_spec(_NBLK, 1)])

    w_flat = w_arr[:, 0]
    hit_mask = jnp.ones((n,), dtype=bool)
    edge_mask = w_flat > 0.5
    return (w_flat.reshape(e, 1), hh, beta, pp, hit_mask, edge_mask)
